# Initial kernel scaffold; baseline (speedup 1.0000x reference)
import jax
import jax.numpy as jnp
from jax.experimental import pallas as pl


def kernel(x, edge_index, edge_attr, W_enc, b_enc, eW00, eb00, eW01, eb01, eW02, eb02, eW10, eb10, eW11, eb11, eW12, eb12, eW20, eb20, eW21, eb21, eW22, eb22, nW00, nb00, nW01, nb01, nW02, nb02, nW10, nb10, nW11, nb11, nW12, nb12, oW0, ob0, oW1, ob1, oW2, ob2):
    raise NotImplementedError("write your pallas kernel here")


# speedup vs baseline: 2.2481x; 2.2481x over previous
import functools

import jax
import jax.numpy as jnp
from jax import lax
from jax.experimental import pallas as pl
from jax.experimental.pallas import tpu as pltpu
from jax.experimental.pallas import tpu_sc as plsc

N = 10000
E = 320000
H = 128

NC = 2
NS = 16
NW = NC * NS
EPW = E // NW
K = 80
NCHUNK = EPW // K
RPW = 624
TAIL0 = RPW * NS
TAILN = N - TAIL0

_F32 = jnp.float32


@functools.cache
def _mesh():
    return plsc.VectorSubcoreMesh(core_axis_name="c", subcore_axis_name="s",
                                  num_cores=NC, num_subcores=NS)


def _gather_sum_body(a_hbm, b_hbm, s_hbm, e_hbm, out_hbm,
                     sidx, eidx, arows, brows, sem_a, sem_b):
    wid = lax.axis_index("s") * NC + lax.axis_index("c")
    base_w = wid * EPW

    def chunk(ci, carry):
        base = base_w + ci * K
        pltpu.sync_copy(s_hbm.at[pl.ds(base, K)], sidx)
        pltpu.sync_copy(e_hbm.at[pl.ds(base, K)], eidx)
        ca = pltpu.async_copy(a_hbm.at[sidx], arows, sem_a)
        cb = pltpu.async_copy(b_hbm.at[eidx], brows, sem_b)
        ca.wait()
        cb.wait()

        def addrow(r, c2):
            for c8 in range(8):
                sl = pl.ds(c8 * 16, 16)
                arows[r, sl] = arows[r, sl] + brows[r, sl]
            return c2

        lax.fori_loop(0, K, addrow, 0)
        pltpu.sync_copy(arows, out_hbm.at[pl.ds(base, K)])
        return carry

    lax.fori_loop(0, NCHUNK, chunk, 0)


def _gather_sum(a, b, s, e):
    return pl.kernel(
        _gather_sum_body,
        out_type=jax.ShapeDtypeStruct((E, H), _F32),
        mesh=_mesh(),
        scratch_types=[
            pltpu.VMEM((K,), jnp.int32),
            pltpu.VMEM((K,), jnp.int32),
            pltpu.VMEM((K, H), _F32),
            pltpu.VMEM((K, H), _F32),
            pltpu.SemaphoreType.DMA,
            pltpu.SemaphoreType.DMA,
        ],
    )(a, b, s, e)


def _scatter_body(h_hbm, s_hbm, e_hbm, ev_hbm, z_hbm, out_hbm,
                  sidx, eidx, ev, hrows, acc, sem_h):
    cid = lax.axis_index("c")
    sid = lax.axis_index("s")
    wid = sid * NC + cid
    pltpu.sync_copy(z_hbm.at[pl.ds(sid * RPW, RPW)],
                    acc.at[pl.ds(sid * RPW, RPW)])

    @pl.when(sid == NS - 1)
    def _zero_tail():
        pltpu.sync_copy(z_hbm.at[pl.ds(TAIL0, TAILN)],
                        acc.at[pl.ds(TAIL0, TAILN)])

    plsc.subcore_barrier()

    base_w = wid * EPW

    def chunk(ci, carry):
        base = base_w + ci * K
        pltpu.sync_copy(s_hbm.at[pl.ds(base, K)], sidx)
        pltpu.sync_copy(e_hbm.at[pl.ds(base, K)], eidx)
        pltpu.sync_copy(ev_hbm.at[pl.ds(base, K)], ev.at[pl.ds(0, K)])
        pltpu.async_copy(h_hbm.at[eidx], hrows, sem_h).wait()

        def mulrow(r, c2):
            bc = ev[pl.ds(r, 16)][0]
            for c8 in range(8):
                sl = pl.ds(c8 * 16, 16)
                hrows[r, sl] = hrows[r, sl] * bc
            return c2

        lax.fori_loop(0, K, mulrow, 0)
        pltpu.sync_copy(hrows, acc.at[sidx], add=True)
        return carry

    lax.fori_loop(0, NCHUNK, chunk, 0)
    plsc.subcore_barrier()
    pltpu.sync_copy(acc.at[pl.ds(sid * RPW, RPW)],
                    out_hbm.at[cid, pl.ds(sid * RPW, RPW)])

    @pl.when(sid == NS - 1)
    def _write_tail():
        pltpu.sync_copy(acc.at[pl.ds(TAIL0, TAILN)],
                        out_hbm.at[cid, pl.ds(TAIL0, TAILN)])


def _scatter(h, s, e, ev, zeros_nh):
    return pl.kernel(
        _scatter_body,
        out_type=jax.ShapeDtypeStruct((NC, N, H), _F32),
        mesh=_mesh(),
        scratch_types=[
            pltpu.VMEM((K,), jnp.int32),
            pltpu.VMEM((K,), jnp.int32),
            pltpu.VMEM((K + 16,), _F32),
            pltpu.VMEM((K, H), _F32),
            pltpu.VMEM_SHARED((N, H), _F32),
            pltpu.SemaphoreType.DMA,
        ],
    )(h, s, e, ev, zeros_nh)


def _degree_body(s_hbm, z_hbm, out_hbm, sidx, ones_v, acc):
    cid = lax.axis_index("c")
    sid = lax.axis_index("s")
    wid = sid * NC + cid

    def fill(r, c2):
        ones_v[r, :] = jnp.ones((16,), _F32)
        return c2

    lax.fori_loop(0, K, fill, 0)
    pltpu.sync_copy(z_hbm.at[pl.ds(sid * RPW, RPW)],
                    acc.at[pl.ds(sid * RPW, RPW)])

    @pl.when(sid == NS - 1)
    def _zero_tail():
        pltpu.sync_copy(z_hbm.at[pl.ds(TAIL0, TAILN)],
                        acc.at[pl.ds(TAIL0, TAILN)])

    plsc.subcore_barrier()

    base_w = wid * EPW

    def chunk(ci, carry):
        base = base_w + ci * K
        pltpu.sync_copy(s_hbm.at[pl.ds(base, K)], sidx)
        pltpu.sync_copy(ones_v, acc.at[sidx], add=True)
        return carry

    lax.fori_loop(0, NCHUNK, chunk, 0)
    plsc.subcore_barrier()
    pltpu.sync_copy(acc.at[pl.ds(sid * RPW, RPW)],
                    out_hbm.at[cid, pl.ds(sid * RPW, RPW)])

    @pl.when(sid == NS - 1)
    def _write_tail():
        pltpu.sync_copy(acc.at[pl.ds(TAIL0, TAILN)],
                        out_hbm.at[cid, pl.ds(TAIL0, TAILN)])


def _degree(s, zeros_n16):
    return pl.kernel(
        _degree_body,
        out_type=jax.ShapeDtypeStruct((NC, N, 16), _F32),
        mesh=_mesh(),
        scratch_types=[
            pltpu.VMEM((K,), jnp.int32),
            pltpu.VMEM((K, 16), _F32),
            pltpu.VMEM_SHARED((N, 16), _F32),
        ],
    )(s, zeros_n16)


_NB = 1000
_EB = 3200


def _full(shape):
    return pl.BlockSpec(shape, lambda i: (0, 0))


def _prep_body(x_ref, we_ref, be_ref, wa_ref, ba_ref, wb_ref,
               h_ref, a_ref, b_ref):
    h = jnp.dot(x_ref[...], we_ref[...], preferred_element_type=_F32) + be_ref[...]
    h_ref[...] = h
    a_ref[...] = jnp.dot(h, wa_ref[...], preferred_element_type=_F32) + ba_ref[...]
    b_ref[...] = jnp.dot(h, wb_ref[...], preferred_element_type=_F32)


def _prep(x_pad, we_pad, be, wa, ba, wb):
    return pl.pallas_call(
        _prep_body,
        grid=(N // _NB,),
        in_specs=[
            pl.BlockSpec((_NB, 8), lambda i: (i, 0)),
            _full((8, H)), _full((1, H)), _full((H, H)), _full((1, H)),
            _full((H, H)),
        ],
        out_specs=[pl.BlockSpec((_NB, H), lambda i: (i, 0))] * 3,
        out_shape=[jax.ShapeDtypeStruct((N, H), _F32)] * 3,
    )(x_pad, we_pad, be, wa, ba, wb)


def _edge_body(g_ref, ea_ref, wc_ref, w1_ref, b1_ref, w2r_ref, b2_ref, e_ref):
    c = jnp.dot(ea_ref[...], wc_ref[...], preferred_element_type=_F32)
    y0 = jnp.maximum(g_ref[...] + c, 0.0)
    y1 = jnp.maximum(
        jnp.dot(y0, w1_ref[...], preferred_element_type=_F32) + b1_ref[...], 0.0)
    t = jnp.sum(y1 * w2r_ref[...], axis=1, keepdims=True) + b2_ref[0:1, 0:1]
    e_ref[...] = jax.nn.sigmoid(jnp.maximum(t, 0.0))


def _edge(g, ea_pad, wc_pad, w1, b1, w2r, b2b):
    return pl.pallas_call(
        _edge_body,
        grid=(E // _EB,),
        in_specs=[
            pl.BlockSpec((_EB, H), lambda i: (i, 0)),
            pl.BlockSpec((_EB, 8), lambda i: (i, 0)),
            _full((8, H)), _full((H, H)), _full((1, H)), _full((1, H)),
            _full((1, H)),
        ],
        out_specs=pl.BlockSpec((_EB, 1), lambda i: (i, 0)),
        out_shape=jax.ShapeDtypeStruct((E, 1), _F32),
    )(g, ea_pad, wc_pad, w1, b1, w2r, b2b)


def _node_body(h_ref, p0_ref, p1_ref, d0_ref, d1_ref,
               w0a_ref, w0b_ref, w0c_ref, b0_ref, w1_ref, b1_ref,
               w2_ref, b2_ref, wa_ref, ba_ref, wb_ref,
               hn_ref, a_ref, b_ref):
    add = p0_ref[...] + p1_ref[...]
    deg = d0_ref[...] + d1_ref[...]
    mean = add * (1.0 / deg[:, 0:1])
    h = h_ref[...]
    ni = (jnp.dot(h, w0a_ref[...], preferred_element_type=_F32)
          + jnp.dot(add, w0b_ref[...], preferred_element_type=_F32)
          + jnp.dot(mean, w0c_ref[...], preferred_element_type=_F32)
          + b0_ref[...])
    y = jnp.maximum(ni, 0.0)
    y = jnp.maximum(jnp.dot(y, w1_ref[...], preferred_element_type=_F32)
                    + b1_ref[...], 0.0)
    y = jnp.maximum(jnp.dot(y, w2_ref[...], preferred_element_type=_F32)
                    + b2_ref[...], 0.0)
    hn = y + h
    hn_ref[...] = hn
    a_ref[...] = jnp.dot(hn, wa_ref[...], preferred_element_type=_F32) + ba_ref[...]
    b_ref[...] = jnp.dot(hn, wb_ref[...], preferred_element_type=_F32)


def _node(h, p0, p1, d0, d1, w0a, w0b, w0c, b0, w1, b1, w2, b2, wa, ba, wb):
    return pl.pallas_call(
        _node_body,
        grid=(N // _NB,),
        in_specs=[
            pl.BlockSpec((_NB, H), lambda i: (i, 0)),
            pl.BlockSpec((_NB, H), lambda i: (i, 0)),
            pl.BlockSpec((_NB, H), lambda i: (i, 0)),
            pl.BlockSpec((_NB, 16), lambda i: (i, 0)),
            pl.BlockSpec((_NB, 16), lambda i: (i, 0)),
            _full((H, H)), _full((H, H)), _full((H, H)), _full((1, H)),
            _full((H, H)), _full((1, H)), _full((H, H)), _full((1, H)),
            _full((H, H)), _full((1, H)), _full((H, H)),
        ],
        out_specs=[pl.BlockSpec((_NB, H), lambda i: (i, 0))] * 3,
        out_shape=[jax.ShapeDtypeStruct((N, H), _F32)] * 3,
    )(h, p0, p1, d0, d1, w0a, w0b, w0c, b0, w1, b1, w2, b2, wa, ba, wb)


def _out_body(h_ref, p0_ref, p1_ref, d0_ref, d1_ref,
              w0a_ref, w0b_ref, w0c_ref, b0_ref, w1_ref, b1_ref,
              w2r_ref, b2_ref, o_ref):
    add = p0_ref[...] + p1_ref[...]
    deg = d0_ref[...] + d1_ref[...]
    mean = add * (1.0 / deg[:, 0:1])
    h = h_ref[...]
    ni = (jnp.dot(h, w0a_ref[...], preferred_element_type=_F32)
          + jnp.dot(add, w0b_ref[...], preferred_element_type=_F32)
          + jnp.dot(mean, w0c_ref[...], preferred_element_type=_F32)
          + b0_ref[...])
    y = jnp.maximum(ni, 0.0)
    y = jnp.maximum(jnp.dot(y, w1_ref[...], preferred_element_type=_F32)
                    + b1_ref[...], 0.0)
    t = jnp.sum(y * w2r_ref[...], axis=1, keepdims=True) + b2_ref[0:1, 0:1]
    o_ref[...] = jax.nn.sigmoid(t)


def _out(h, p0, p1, d0, d1, w0a, w0b, w0c, b0, w1, b1, w2r, b2b):
    return pl.pallas_call(
        _out_body,
        grid=(N // _NB,),
        in_specs=[
            pl.BlockSpec((_NB, H), lambda i: (i, 0)),
            pl.BlockSpec((_NB, H), lambda i: (i, 0)),
            pl.BlockSpec((_NB, H), lambda i: (i, 0)),
            pl.BlockSpec((_NB, 16), lambda i: (i, 0)),
            pl.BlockSpec((_NB, 16), lambda i: (i, 0)),
            _full((H, H)), _full((H, H)), _full((H, H)), _full((1, H)),
            _full((H, H)), _full((1, H)), _full((1, H)), _full((1, H)),
        ],
        out_specs=pl.BlockSpec((_NB, 1), lambda i: (i, 0)),
        out_shape=jax.ShapeDtypeStruct((N, 1), _F32),
    )(h, p0, p1, d0, d1, w0a, w0b, w0c, b0, w1, b1, w2r, b2b)


def kernel(x, edge_index, edge_attr, W_enc, b_enc,
           eW00, eb00, eW01, eb01, eW02, eb02,
           eW10, eb10, eW11, eb11, eW12, eb12,
           eW20, eb20, eW21, eb21, eW22, eb22,
           nW00, nb00, nW01, nb01, nW02, nb02,
           nW10, nb10, nW11, nb11, nW12, nb12,
           oW0, ob0, oW1, ob1, oW2, ob2):
    start = edge_index[0]
    end = edge_index[1]
    x_pad = jnp.pad(x, ((0, 0), (0, 5)))
    we_pad = jnp.pad(W_enc, ((0, 5), (0, 0)))
    ea_pad = jnp.pad(edge_attr, ((0, 0), (0, 6)))
    zeros_nh = jnp.zeros((N, H), _F32)
    zeros_n16 = jnp.zeros((N, 16), _F32)

    eW0 = [eW00, eW10, eW20]
    eb0 = [eb00, eb10, eb20]
    eW1 = [eW01, eW11, eW21]
    eb1 = [eb01, eb11, eb21]
    eW2 = [eW02, eW12, eW22]
    eb2 = [eb02, eb12, eb22]
    nW = [[nW00, nW01, nW02], [nW10, nW11, nW12], [oW0, oW1, oW2]]
    nb = [[nb00, nb01, nb02], [nb10, nb11, nb12], [ob0, ob1, ob2]]

    def w0_split(i):
        w0 = eW0[i]
        return (w0[:H], w0[H:2 * H],
                jnp.pad(w0[2 * H:], ((0, 6), (0, 0))),
                eb0[i].reshape(1, H))

    deg2 = _degree(start, zeros_n16)
    d0, d1 = deg2[0], deg2[1]

    wa0, wb0, _, ba0 = w0_split(0)
    h, a, b = _prep(x_pad, we_pad, b_enc.reshape(1, H), wa0, ba0, wb0)

    out = None
    for i in range(3):
        _, _, wc_pad, _ = w0_split(i)
        w2r = eW2[i].reshape(1, H)
        b2b = jnp.broadcast_to(eb2[i].reshape(1, 1), (1, H))
        g = _gather_sum(a, b, start, end)
        ev = _edge(g, ea_pad, wc_pad, eW1[i], eb1[i].reshape(1, H), w2r, b2b)
        parts = _scatter(h, start, end, ev.reshape(E), zeros_nh)
        p0, p1 = parts[0], parts[1]
        w0, w1, w2 = nW[i]
        c0, c1, c2 = nb[i]
        n0a, n0b, n0c = w0[:H], w0[H:2 * H], w0[2 * H:]
        if i < 2:
            wan, wbn, _, ban = w0_split(i + 1)
            h, a, b = _node(h, p0, p1, d0, d1, n0a, n0b, n0c,
                            c0.reshape(1, H), w1, c1.reshape(1, H),
                            w2, c2.reshape(1, H), wan, ban, wbn)
        else:
            w2r_o = w2.reshape(1, H)
            b2b_o = jnp.broadcast_to(c2.reshape(1, 1), (1, H))
            out = _out(h, p0, p1, d0, d1, n0a, n0b, n0c,
                       c0.reshape(1, H), w1, c1.reshape(1, H), w2r_o, b2b_o)
    return out


# speedup vs baseline: 3.9107x; 1.7395x over previous
import functools

import jax
import jax.numpy as jnp
from jax import lax
from jax.experimental import pallas as pl
from jax.experimental.pallas import tpu as pltpu
from jax.experimental.pallas import tpu_sc as plsc

N = 10000
E = 320000
H = 128

NC = 2
NS = 16
NW = NC * NS
EPW = E // NW
K = 80
NCHUNK = EPW // K
RPW = 624
TAIL0 = RPW * NS
TAILN = N - TAIL0

_F32 = jnp.float32


@functools.cache
def _mesh():
    return plsc.VectorSubcoreMesh(core_axis_name="c", subcore_axis_name="s",
                                  num_cores=NC, num_subcores=NS)


def _gather_sum_body(a_hbm, b_hbm, s_hbm, e_hbm, out_hbm,
                     sidx_all, eidx_all, ar0, br0, ar1, br1,
                     sem_a0, sem_b0, sem_o0, sem_a1, sem_b1, sem_o1):
    wid = lax.axis_index("s") * NC + lax.axis_index("c")
    base_w = wid * EPW
    pltpu.sync_copy(s_hbm.at[pl.ds(base_w, EPW)], sidx_all)
    pltpu.sync_copy(e_hbm.at[pl.ds(base_w, EPW)], eidx_all)

    bufs = ((ar0, br0, sem_a0, sem_b0, sem_o0),
            (ar1, br1, sem_a1, sem_b1, sem_o1))

    def issue(ci, b):
        ar, br, sa, sb, _ = bufs[b]
        pltpu.async_copy(a_hbm.at[sidx_all.at[pl.ds(ci * K, K)]], ar, sa)
        pltpu.async_copy(b_hbm.at[eidx_all.at[pl.ds(ci * K, K)]], br, sb)

    def wait_gather(ci, b):
        ar, br, sa, sb, _ = bufs[b]
        pltpu.make_async_copy(
            a_hbm.at[sidx_all.at[pl.ds(ci * K, K)]], ar, sa).wait()
        pltpu.make_async_copy(
            b_hbm.at[eidx_all.at[pl.ds(ci * K, K)]], br, sb).wait()

    def drain_out(ci, b):
        ar = bufs[b][0]
        so = bufs[b][4]
        pltpu.make_async_copy(
            ar, out_hbm.at[pl.ds(base_w + ci * K, K)], so).wait()

    issue(0, 0)

    def body(ci, carry):
        for par in (0, 1):
            @pl.when(ci % 2 == par)
            def _process():
                cb, nb = par, 1 - par

                @pl.when(ci >= 1)
                def _drain():
                    drain_out(ci - 1, nb)

                @pl.when(ci + 1 < NCHUNK)
                def _issue():
                    issue(ci + 1, nb)

                wait_gather(ci, cb)
                ar, br = bufs[cb][0], bufs[cb][1]
                so = bufs[cb][4]

                def addrow(r, c2):
                    for c8 in range(8):
                        sl = pl.ds(c8 * 16, 16)
                        ar[r, sl] = ar[r, sl] + br[r, sl]
                    return c2

                lax.fori_loop(0, K, addrow, 0)
                pltpu.async_copy(ar, out_hbm.at[pl.ds(base_w + ci * K, K)], so)
        return carry

    lax.fori_loop(0, NCHUNK, body, 0)
    drain_out(NCHUNK - 1, (NCHUNK - 1) % 2)


def _gather_sum(a, b, s, e):
    return pl.kernel(
        _gather_sum_body,
        out_type=jax.ShapeDtypeStruct((E, H), _F32),
        mesh=_mesh(),
        scratch_types=[
            pltpu.VMEM((EPW,), jnp.int32),
            pltpu.VMEM((EPW,), jnp.int32),
            pltpu.VMEM((K, H), _F32),
            pltpu.VMEM((K, H), _F32),
            pltpu.VMEM((K, H), _F32),
            pltpu.VMEM((K, H), _F32),
            pltpu.SemaphoreType.DMA,
            pltpu.SemaphoreType.DMA,
            pltpu.SemaphoreType.DMA,
            pltpu.SemaphoreType.DMA,
            pltpu.SemaphoreType.DMA,
            pltpu.SemaphoreType.DMA,
        ],
    )(a, b, s, e)


def _scatter_body(h_hbm, s_hbm, e_hbm, ev_hbm, z_hbm, out_hbm,
                  sidx_all, eidx_all, ev_all, hr0, hr1, st0, st1,
                  sem_h0, sem_c0, sem_h1, sem_c1, acc):
    cid = lax.axis_index("c")
    sid = lax.axis_index("s")
    wid = sid * NC + cid
    base_w = wid * EPW
    pltpu.sync_copy(z_hbm.at[pl.ds(sid * RPW, RPW)],
                    acc.at[pl.ds(sid * RPW, RPW)])

    @pl.when(sid == NS - 1)
    def _zero_tail():
        pltpu.sync_copy(z_hbm.at[pl.ds(TAIL0, TAILN)],
                        acc.at[pl.ds(TAIL0, TAILN)])

    pltpu.sync_copy(s_hbm.at[pl.ds(base_w, EPW)], sidx_all)
    pltpu.sync_copy(e_hbm.at[pl.ds(base_w, EPW)], eidx_all)
    pltpu.sync_copy(ev_hbm.at[pl.ds(base_w, EPW)], ev_all.at[pl.ds(0, EPW)])
    plsc.subcore_barrier()

    bufs = ((hr0, st0, sem_h0, sem_c0), (hr1, st1, sem_h1, sem_c1))

    def issue(ci, b):
        hr, _, sh, _ = bufs[b]
        pltpu.async_copy(h_hbm.at[eidx_all.at[pl.ds(ci * K, K)]], hr, sh)

    def wait_gather(ci, b):
        hr, _, sh, _ = bufs[b]
        pltpu.make_async_copy(
            h_hbm.at[eidx_all.at[pl.ds(ci * K, K)]], hr, sh).wait()

    def drain_scatter(b):
        hr, st, _, sc = bufs[b]
        pltpu.make_async_copy(hr, acc.at[st], sc).wait()

    issue(0, 0)

    def body(ci, carry):
        for par in (0, 1):
            @pl.when(ci % 2 == par)
            def _process():
                cb, nb = par, 1 - par

                @pl.when(ci >= 1)
                def _drain():
                    drain_scatter(nb)

                @pl.when(ci + 1 < NCHUNK)
                def _issue():
                    issue(ci + 1, nb)

                wait_gather(ci, cb)
                hr, st, _, sc = bufs[cb]
                cb_base = ci * K

                def mulrow(r, c2):
                    bc = ev_all[pl.ds(cb_base + r, 16)][0]
                    for c8 in range(8):
                        sl = pl.ds(c8 * 16, 16)
                        hr[r, sl] = hr[r, sl] * bc
                    return c2

                lax.fori_loop(0, K, mulrow, 0)
                for k16 in range(K // 16):
                    st[pl.ds(16 * k16, 16)] = (
                        sidx_all[pl.ds(cb_base + 16 * k16, 16)])
                pltpu.async_copy(hr, acc.at[st], sc, add=True)
        return carry

    lax.fori_loop(0, NCHUNK, body, 0)
    drain_scatter((NCHUNK - 1) % 2)
    plsc.subcore_barrier()
    pltpu.sync_copy(acc.at[pl.ds(sid * RPW, RPW)],
                    out_hbm.at[cid, pl.ds(sid * RPW, RPW)])

    @pl.when(sid == NS - 1)
    def _write_tail():
        pltpu.sync_copy(acc.at[pl.ds(TAIL0, TAILN)],
                        out_hbm.at[cid, pl.ds(TAIL0, TAILN)])


def _scatter(h, s, e, ev, zeros_nh):
    return pl.kernel(
        _scatter_body,
        out_type=jax.ShapeDtypeStruct((NC, N, H), _F32),
        mesh=_mesh(),
        scratch_types=[
            pltpu.VMEM((EPW,), jnp.int32),
            pltpu.VMEM((EPW,), jnp.int32),
            pltpu.VMEM((EPW + 16,), _F32),
            pltpu.VMEM((K, H), _F32),
            pltpu.VMEM((K, H), _F32),
            pltpu.VMEM((K,), jnp.int32),
            pltpu.VMEM((K,), jnp.int32),
            pltpu.SemaphoreType.DMA,
            pltpu.SemaphoreType.DMA,
            pltpu.SemaphoreType.DMA,
            pltpu.SemaphoreType.DMA,
            pltpu.VMEM_SHARED((N, H), _F32),
        ],
    )(h, s, e, ev, zeros_nh)


def _degree_body(s_hbm, z_hbm, out_hbm, sidx_all, st, ones_v, acc):
    cid = lax.axis_index("c")
    sid = lax.axis_index("s")
    wid = sid * NC + cid
    base_w = wid * EPW

    def fill(r, c2):
        ones_v[r, :] = jnp.ones((16,), _F32)
        return c2

    lax.fori_loop(0, K, fill, 0)
    pltpu.sync_copy(z_hbm.at[pl.ds(sid * RPW, RPW)],
                    acc.at[pl.ds(sid * RPW, RPW)])

    @pl.when(sid == NS - 1)
    def _zero_tail():
        pltpu.sync_copy(z_hbm.at[pl.ds(TAIL0, TAILN)],
                        acc.at[pl.ds(TAIL0, TAILN)])

    pltpu.sync_copy(s_hbm.at[pl.ds(base_w, EPW)], sidx_all)
    plsc.subcore_barrier()

    def chunk(ci, carry):
        for k16 in range(K // 16):
            st[pl.ds(16 * k16, 16)] = sidx_all[pl.ds(ci * K + 16 * k16, 16)]
        pltpu.sync_copy(ones_v, acc.at[st], add=True)
        return carry

    lax.fori_loop(0, NCHUNK, chunk, 0)
    plsc.subcore_barrier()
    pltpu.sync_copy(acc.at[pl.ds(sid * RPW, RPW)],
                    out_hbm.at[cid, pl.ds(sid * RPW, RPW)])

    @pl.when(sid == NS - 1)
    def _write_tail():
        pltpu.sync_copy(acc.at[pl.ds(TAIL0, TAILN)],
                        out_hbm.at[cid, pl.ds(TAIL0, TAILN)])


def _degree(s, zeros_n16):
    return pl.kernel(
        _degree_body,
        out_type=jax.ShapeDtypeStruct((NC, N, 16), _F32),
        mesh=_mesh(),
        scratch_types=[
            pltpu.VMEM((EPW,), jnp.int32),
            pltpu.VMEM((K,), jnp.int32),
            pltpu.VMEM((K, 16), _F32),
            pltpu.VMEM_SHARED((N, 16), _F32),
        ],
    )(s, zeros_n16)


_NB = 1000
_EB = 3200


def _full(shape):
    return pl.BlockSpec(shape, lambda i: (0, 0))


def _prep_body(x_ref, we_ref, be_ref, wa_ref, ba_ref, wb_ref,
               h_ref, a_ref, b_ref):
    h = jnp.dot(x_ref[...], we_ref[...], preferred_element_type=_F32) + be_ref[...]
    h_ref[...] = h
    a_ref[...] = jnp.dot(h, wa_ref[...], preferred_element_type=_F32) + ba_ref[...]
    b_ref[...] = jnp.dot(h, wb_ref[...], preferred_element_type=_F32)


def _prep(x_pad, we_pad, be, wa, ba, wb):
    return pl.pallas_call(
        _prep_body,
        grid=(N // _NB,),
        in_specs=[
            pl.BlockSpec((_NB, 8), lambda i: (i, 0)),
            _full((8, H)), _full((1, H)), _full((H, H)), _full((1, H)),
            _full((H, H)),
        ],
        out_specs=[pl.BlockSpec((_NB, H), lambda i: (i, 0))] * 3,
        out_shape=[jax.ShapeDtypeStruct((N, H), _F32)] * 3,
    )(x_pad, we_pad, be, wa, ba, wb)


def _edge_body(g_ref, ea_ref, wc_ref, w1_ref, b1_ref, w2r_ref, b2_ref, e_ref):
    c = jnp.dot(ea_ref[...], wc_ref[...], preferred_element_type=_F32)
    y0 = jnp.maximum(g_ref[...] + c, 0.0)
    y1 = jnp.maximum(
        jnp.dot(y0, w1_ref[...], preferred_element_type=_F32) + b1_ref[...], 0.0)
    t = jnp.sum(y1 * w2r_ref[...], axis=1, keepdims=True) + b2_ref[0:1, 0:1]
    e_ref[...] = jax.nn.sigmoid(jnp.maximum(t, 0.0))


def _edge(g, ea_pad, wc_pad, w1, b1, w2r, b2b):
    return pl.pallas_call(
        _edge_body,
        grid=(E // _EB,),
        in_specs=[
            pl.BlockSpec((_EB, H), lambda i: (i, 0)),
            pl.BlockSpec((_EB, 8), lambda i: (i, 0)),
            _full((8, H)), _full((H, H)), _full((1, H)), _full((1, H)),
            _full((1, H)),
        ],
        out_specs=pl.BlockSpec((_EB, 1), lambda i: (i, 0)),
        out_shape=jax.ShapeDtypeStruct((E, 1), _F32),
    )(g, ea_pad, wc_pad, w1, b1, w2r, b2b)


def _node_body(h_ref, p0_ref, p1_ref, d0_ref, d1_ref,
               w0a_ref, w0b_ref, w0c_ref, b0_ref, w1_ref, b1_ref,
               w2_ref, b2_ref, wa_ref, ba_ref, wb_ref,
               hn_ref, a_ref, b_ref):
    add = p0_ref[...] + p1_ref[...]
    deg = d0_ref[...] + d1_ref[...]
    mean = add * (1.0 / deg[:, 0:1])
    h = h_ref[...]
    ni = (jnp.dot(h, w0a_ref[...], preferred_element_type=_F32)
          + jnp.dot(add, w0b_ref[...], preferred_element_type=_F32)
          + jnp.dot(mean, w0c_ref[...], preferred_element_type=_F32)
          + b0_ref[...])
    y = jnp.maximum(ni, 0.0)
    y = jnp.maximum(jnp.dot(y, w1_ref[...], preferred_element_type=_F32)
                    + b1_ref[...], 0.0)
    y = jnp.maximum(jnp.dot(y, w2_ref[...], preferred_element_type=_F32)
                    + b2_ref[...], 0.0)
    hn = y + h
    hn_ref[...] = hn
    a_ref[...] = jnp.dot(hn, wa_ref[...], preferred_element_type=_F32) + ba_ref[...]
    b_ref[...] = jnp.dot(hn, wb_ref[...], preferred_element_type=_F32)


def _node(h, p0, p1, d0, d1, w0a, w0b, w0c, b0, w1, b1, w2, b2, wa, ba, wb):
    return pl.pallas_call(
        _node_body,
        grid=(N // _NB,),
        in_specs=[
            pl.BlockSpec((_NB, H), lambda i: (i, 0)),
            pl.BlockSpec((_NB, H), lambda i: (i, 0)),
            pl.BlockSpec((_NB, H), lambda i: (i, 0)),
            pl.BlockSpec((_NB, 16), lambda i: (i, 0)),
            pl.BlockSpec((_NB, 16), lambda i: (i, 0)),
            _full((H, H)), _full((H, H)), _full((H, H)), _full((1, H)),
            _full((H, H)), _full((1, H)), _full((H, H)), _full((1, H)),
            _full((H, H)), _full((1, H)), _full((H, H)),
        ],
        out_specs=[pl.BlockSpec((_NB, H), lambda i: (i, 0))] * 3,
        out_shape=[jax.ShapeDtypeStruct((N, H), _F32)] * 3,
    )(h, p0, p1, d0, d1, w0a, w0b, w0c, b0, w1, b1, w2, b2, wa, ba, wb)


def _out_body(h_ref, p0_ref, p1_ref, d0_ref, d1_ref,
              w0a_ref, w0b_ref, w0c_ref, b0_ref, w1_ref, b1_ref,
              w2r_ref, b2_ref, o_ref):
    add = p0_ref[...] + p1_ref[...]
    deg = d0_ref[...] + d1_ref[...]
    mean = add * (1.0 / deg[:, 0:1])
    h = h_ref[...]
    ni = (jnp.dot(h, w0a_ref[...], preferred_element_type=_F32)
          + jnp.dot(add, w0b_ref[...], preferred_element_type=_F32)
          + jnp.dot(mean, w0c_ref[...], preferred_element_type=_F32)
          + b0_ref[...])
    y = jnp.maximum(ni, 0.0)
    y = jnp.maximum(jnp.dot(y, w1_ref[...], preferred_element_type=_F32)
                    + b1_ref[...], 0.0)
    t = jnp.sum(y * w2r_ref[...], axis=1, keepdims=True) + b2_ref[0:1, 0:1]
    o_ref[...] = jax.nn.sigmoid(t)


def _out(h, p0, p1, d0, d1, w0a, w0b, w0c, b0, w1, b1, w2r, b2b):
    return pl.pallas_call(
        _out_body,
        grid=(N // _NB,),
        in_specs=[
            pl.BlockSpec((_NB, H), lambda i: (i, 0)),
            pl.BlockSpec((_NB, H), lambda i: (i, 0)),
            pl.BlockSpec((_NB, H), lambda i: (i, 0)),
            pl.BlockSpec((_NB, 16), lambda i: (i, 0)),
            pl.BlockSpec((_NB, 16), lambda i: (i, 0)),
            _full((H, H)), _full((H, H)), _full((H, H)), _full((1, H)),
            _full((H, H)), _full((1, H)), _full((1, H)), _full((1, H)),
        ],
        out_specs=pl.BlockSpec((_NB, 1), lambda i: (i, 0)),
        out_shape=jax.ShapeDtypeStruct((N, 1), _F32),
    )(h, p0, p1, d0, d1, w0a, w0b, w0c, b0, w1, b1, w2r, b2b)


def kernel(x, edge_index, edge_attr, W_enc, b_enc,
           eW00, eb00, eW01, eb01, eW02, eb02,
           eW10, eb10, eW11, eb11, eW12, eb12,
           eW20, eb20, eW21, eb21, eW22, eb22,
           nW00, nb00, nW01, nb01, nW02, nb02,
           nW10, nb10, nW11, nb11, nW12, nb12,
           oW0, ob0, oW1, ob1, oW2, ob2):
    start = edge_index[0]
    end = edge_index[1]
    x_pad = jnp.pad(x, ((0, 0), (0, 5)))
    we_pad = jnp.pad(W_enc, ((0, 5), (0, 0)))
    ea_pad = jnp.pad(edge_attr, ((0, 0), (0, 6)))
    zeros_nh = jnp.zeros((N, H), _F32)
    zeros_n16 = jnp.zeros((N, 16), _F32)

    eW0 = [eW00, eW10, eW20]
    eb0 = [eb00, eb10, eb20]
    eW1 = [eW01, eW11, eW21]
    eb1 = [eb01, eb11, eb21]
    eW2 = [eW02, eW12, eW22]
    eb2 = [eb02, eb12, eb22]
    nW = [[nW00, nW01, nW02], [nW10, nW11, nW12], [oW0, oW1, oW2]]
    nb = [[nb00, nb01, nb02], [nb10, nb11, nb12], [ob0, ob1, ob2]]

    def w0_split(i):
        w0 = eW0[i]
        return (w0[:H], w0[H:2 * H],
                jnp.pad(w0[2 * H:], ((0, 6), (0, 0))),
                eb0[i].reshape(1, H))

    deg2 = _degree(start, zeros_n16)
    d0, d1 = deg2[0], deg2[1]

    wa0, wb0, _, ba0 = w0_split(0)
    h, a, b = _prep(x_pad, we_pad, b_enc.reshape(1, H), wa0, ba0, wb0)

    out = None
    for i in range(3):
        _, _, wc_pad, _ = w0_split(i)
        w2r = eW2[i].reshape(1, H)
        b2b = jnp.broadcast_to(eb2[i].reshape(1, 1), (1, H))
        g = _gather_sum(a, b, start, end)
        ev = _edge(g, ea_pad, wc_pad, eW1[i], eb1[i].reshape(1, H), w2r, b2b)
        parts = _scatter(h, start, end, ev.reshape(E), zeros_nh)
        p0, p1 = parts[0], parts[1]
        w0, w1, w2 = nW[i]
        c0, c1, c2 = nb[i]
        n0a, n0b, n0c = w0[:H], w0[H:2 * H], w0[2 * H:]
        if i < 2:
            wan, wbn, _, ban = w0_split(i + 1)
            h, a, b = _node(h, p0, p1, d0, d1, n0a, n0b, n0c,
                            c0.reshape(1, H), w1, c1.reshape(1, H),
                            w2, c2.reshape(1, H), wan, ban, wbn)
        else:
            w2r_o = w2.reshape(1, H)
            b2b_o = jnp.broadcast_to(c2.reshape(1, 1), (1, H))
            out = _out(h, p0, p1, d0, d1, n0a, n0b, n0c,
                       c0.reshape(1, H), w1, c1.reshape(1, H), w2r_o, b2b_o)
    return out


# speedup vs baseline: 4.0020x; 1.0233x over previous
import functools

import jax
import jax.numpy as jnp
from jax import lax
from jax.experimental import pallas as pl
from jax.experimental.pallas import tpu as pltpu
from jax.experimental.pallas import tpu_sc as plsc

N = 10000
E = 320000
H = 128

NC = 2
NS = 16
NW = NC * NS
EPW = E // NW
K = 80
NCHUNK = EPW // K
RPW = 624
TAIL0 = RPW * NS
TAILN = N - TAIL0

_F32 = jnp.float32


@functools.cache
def _mesh():
    return plsc.VectorSubcoreMesh(core_axis_name="c", subcore_axis_name="s",
                                  num_cores=NC, num_subcores=NS)


def _gather_sum_body(a_hbm, b_hbm, s_hbm, e_hbm, out_hbm,
                     sidx_all, eidx_all, ar0, br0, ar1, br1, ar2, br2,
                     sem_a0, sem_b0, sem_o0, sem_a1, sem_b1, sem_o1,
                     sem_a2, sem_b2, sem_o2):
    wid = lax.axis_index("s") * NC + lax.axis_index("c")
    base_w = wid * EPW
    pltpu.sync_copy(s_hbm.at[pl.ds(base_w, EPW)], sidx_all)
    pltpu.sync_copy(e_hbm.at[pl.ds(base_w, EPW)], eidx_all)

    bufs = ((ar0, br0, sem_a0, sem_b0, sem_o0),
            (ar1, br1, sem_a1, sem_b1, sem_o1),
            (ar2, br2, sem_a2, sem_b2, sem_o2))

    def issue(ci, b):
        ar, br, sa, sb, _ = bufs[b]
        pltpu.async_copy(a_hbm.at[sidx_all.at[pl.ds(ci * K, K)]], ar, sa)
        pltpu.async_copy(b_hbm.at[eidx_all.at[pl.ds(ci * K, K)]], br, sb)

    def wait_gather(ci, b):
        ar, br, sa, sb, _ = bufs[b]
        pltpu.make_async_copy(
            a_hbm.at[sidx_all.at[pl.ds(ci * K, K)]], ar, sa).wait()
        pltpu.make_async_copy(
            b_hbm.at[eidx_all.at[pl.ds(ci * K, K)]], br, sb).wait()

    def drain_out(ci, b):
        ar = bufs[b][0]
        so = bufs[b][4]
        pltpu.make_async_copy(
            ar, out_hbm.at[pl.ds(base_w + ci * K, K)], so).wait()

    issue(0, 0)
    issue(1, 1)

    def body(ci, carry):
        for par in (0, 1, 2):
            @pl.when(ci % 3 == par)
            def _process():
                cb = par
                nb = (par + 2) % 3

                @pl.when(ci >= 1)
                def _drain():
                    drain_out(ci - 1, nb)

                @pl.when(ci + 2 < NCHUNK)
                def _issue():
                    issue(ci + 2, nb)

                wait_gather(ci, cb)
                ar, br = bufs[cb][0], bufs[cb][1]
                so = bufs[cb][4]

                def addrow(r, c2):
                    for c8 in range(8):
                        sl = pl.ds(c8 * 16, 16)
                        ar[r, sl] = ar[r, sl] + br[r, sl]
                    return c2

                lax.fori_loop(0, K, addrow, 0)
                pltpu.async_copy(ar, out_hbm.at[pl.ds(base_w + ci * K, K)], so)
        return carry

    lax.fori_loop(0, NCHUNK, body, 0)
    drain_out(NCHUNK - 1, (NCHUNK - 1) % 3)


def _gather_sum(a, b, s, e):
    return pl.kernel(
        _gather_sum_body,
        out_type=jax.ShapeDtypeStruct((E, H), _F32),
        mesh=_mesh(),
        scratch_types=[
            pltpu.VMEM((EPW,), jnp.int32),
            pltpu.VMEM((EPW,), jnp.int32),
            pltpu.VMEM((K, H), _F32),
            pltpu.VMEM((K, H), _F32),
            pltpu.VMEM((K, H), _F32),
            pltpu.VMEM((K, H), _F32),
            pltpu.VMEM((K, H), _F32),
            pltpu.VMEM((K, H), _F32),
            pltpu.SemaphoreType.DMA,
            pltpu.SemaphoreType.DMA,
            pltpu.SemaphoreType.DMA,
            pltpu.SemaphoreType.DMA,
            pltpu.SemaphoreType.DMA,
            pltpu.SemaphoreType.DMA,
            pltpu.SemaphoreType.DMA,
            pltpu.SemaphoreType.DMA,
            pltpu.SemaphoreType.DMA,
        ],
    )(a, b, s, e)


def _scatter_body(h_hbm, s_hbm, e_hbm, ev_hbm, z_hbm, out_hbm,
                  sidx_all, eidx_all, ev_all, hr0, hr1, st0, st1,
                  sem_h0, sem_c0, sem_h1, sem_c1, acc):
    cid = lax.axis_index("c")
    sid = lax.axis_index("s")
    wid = sid * NC + cid
    base_w = wid * EPW
    pltpu.sync_copy(z_hbm.at[pl.ds(sid * RPW, RPW)],
                    acc.at[pl.ds(sid * RPW, RPW)])

    @pl.when(sid == NS - 1)
    def _zero_tail():
        pltpu.sync_copy(z_hbm.at[pl.ds(TAIL0, TAILN)],
                        acc.at[pl.ds(TAIL0, TAILN)])

    pltpu.sync_copy(s_hbm.at[pl.ds(base_w, EPW)], sidx_all)
    pltpu.sync_copy(e_hbm.at[pl.ds(base_w, EPW)], eidx_all)
    pltpu.sync_copy(ev_hbm.at[pl.ds(base_w, EPW)], ev_all.at[pl.ds(0, EPW)])
    plsc.subcore_barrier()

    bufs = ((hr0, st0, sem_h0, sem_c0), (hr1, st1, sem_h1, sem_c1))

    def issue(ci, b):
        hr, _, sh, _ = bufs[b]
        pltpu.async_copy(h_hbm.at[eidx_all.at[pl.ds(ci * K, K)]], hr, sh)

    def wait_gather(ci, b):
        hr, _, sh, _ = bufs[b]
        pltpu.make_async_copy(
            h_hbm.at[eidx_all.at[pl.ds(ci * K, K)]], hr, sh).wait()

    def drain_scatter(b):
        hr, st, _, sc = bufs[b]
        pltpu.make_async_copy(hr, acc.at[st], sc).wait()

    issue(0, 0)

    def body(ci, carry):
        for par in (0, 1):
            @pl.when(ci % 2 == par)
            def _process():
                cb, nb = par, 1 - par

                @pl.when(ci >= 1)
                def _drain():
                    drain_scatter(nb)

                @pl.when(ci + 1 < NCHUNK)
                def _issue():
                    issue(ci + 1, nb)

                wait_gather(ci, cb)
                hr, st, _, sc = bufs[cb]
                cb_base = ci * K

                def mulrow(r, c2):
                    bc = ev_all[pl.ds(cb_base + r, 16)][0]
                    for c8 in range(8):
                        sl = pl.ds(c8 * 16, 16)
                        hr[r, sl] = hr[r, sl] * bc
                    return c2

                lax.fori_loop(0, K, mulrow, 0)
                for k16 in range(K // 16):
                    st[pl.ds(16 * k16, 16)] = (
                        sidx_all[pl.ds(cb_base + 16 * k16, 16)])
                pltpu.async_copy(hr, acc.at[st], sc, add=True)
        return carry

    lax.fori_loop(0, NCHUNK, body, 0)
    drain_scatter((NCHUNK - 1) % 2)
    plsc.subcore_barrier()
    pltpu.sync_copy(acc.at[pl.ds(sid * RPW, RPW)],
                    out_hbm.at[cid, pl.ds(sid * RPW, RPW)])

    @pl.when(sid == NS - 1)
    def _write_tail():
        pltpu.sync_copy(acc.at[pl.ds(TAIL0, TAILN)],
                        out_hbm.at[cid, pl.ds(TAIL0, TAILN)])


def _scatter(h, s, e, ev, zeros_nh):
    return pl.kernel(
        _scatter_body,
        out_type=jax.ShapeDtypeStruct((NC, N, H), _F32),
        mesh=_mesh(),
        scratch_types=[
            pltpu.VMEM((EPW,), jnp.int32),
            pltpu.VMEM((EPW,), jnp.int32),
            pltpu.VMEM((EPW + 16,), _F32),
            pltpu.VMEM((K, H), _F32),
            pltpu.VMEM((K, H), _F32),
            pltpu.VMEM((K,), jnp.int32),
            pltpu.VMEM((K,), jnp.int32),
            pltpu.SemaphoreType.DMA,
            pltpu.SemaphoreType.DMA,
            pltpu.SemaphoreType.DMA,
            pltpu.SemaphoreType.DMA,
            pltpu.VMEM_SHARED((N, H), _F32),
        ],
    )(h, s, e, ev, zeros_nh)


def _degree_body(s_hbm, z_hbm, out_hbm, sidx_all, st, ones_v, acc):
    cid = lax.axis_index("c")
    sid = lax.axis_index("s")
    wid = sid * NC + cid
    base_w = wid * EPW

    def fill(r, c2):
        ones_v[r, :] = jnp.ones((16,), _F32)
        return c2

    lax.fori_loop(0, K, fill, 0)
    pltpu.sync_copy(z_hbm.at[pl.ds(sid * RPW, RPW)],
                    acc.at[pl.ds(sid * RPW, RPW)])

    @pl.when(sid == NS - 1)
    def _zero_tail():
        pltpu.sync_copy(z_hbm.at[pl.ds(TAIL0, TAILN)],
                        acc.at[pl.ds(TAIL0, TAILN)])

    pltpu.sync_copy(s_hbm.at[pl.ds(base_w, EPW)], sidx_all)
    plsc.subcore_barrier()

    def chunk(ci, carry):
        for k16 in range(K // 16):
            st[pl.ds(16 * k16, 16)] = sidx_all[pl.ds(ci * K + 16 * k16, 16)]
        pltpu.sync_copy(ones_v, acc.at[st], add=True)
        return carry

    lax.fori_loop(0, NCHUNK, chunk, 0)
    plsc.subcore_barrier()
    pltpu.sync_copy(acc.at[pl.ds(sid * RPW, RPW)],
                    out_hbm.at[cid, pl.ds(sid * RPW, RPW)])

    @pl.when(sid == NS - 1)
    def _write_tail():
        pltpu.sync_copy(acc.at[pl.ds(TAIL0, TAILN)],
                        out_hbm.at[cid, pl.ds(TAIL0, TAILN)])


def _degree(s, zeros_n16):
    return pl.kernel(
        _degree_body,
        out_type=jax.ShapeDtypeStruct((NC, N, 16), _F32),
        mesh=_mesh(),
        scratch_types=[
            pltpu.VMEM((EPW,), jnp.int32),
            pltpu.VMEM((K,), jnp.int32),
            pltpu.VMEM((K, 16), _F32),
            pltpu.VMEM_SHARED((N, 16), _F32),
        ],
    )(s, zeros_n16)


_NB = 1000
_EB = 3200


def _full(shape):
    return pl.BlockSpec(shape, lambda i: (0, 0))


def _prep_body(x_ref, we_ref, be_ref, wa_ref, ba_ref, wb_ref,
               h_ref, a_ref, b_ref):
    h = jnp.dot(x_ref[...], we_ref[...], preferred_element_type=_F32) + be_ref[...]
    h_ref[...] = h
    a_ref[...] = jnp.dot(h, wa_ref[...], preferred_element_type=_F32) + ba_ref[...]
    b_ref[...] = jnp.dot(h, wb_ref[...], preferred_element_type=_F32)


def _prep(x_pad, we_pad, be, wa, ba, wb):
    return pl.pallas_call(
        _prep_body,
        grid=(N // _NB,),
        in_specs=[
            pl.BlockSpec((_NB, 8), lambda i: (i, 0)),
            _full((8, H)), _full((1, H)), _full((H, H)), _full((1, H)),
            _full((H, H)),
        ],
        out_specs=[pl.BlockSpec((_NB, H), lambda i: (i, 0))] * 3,
        out_shape=[jax.ShapeDtypeStruct((N, H), _F32)] * 3,
    )(x_pad, we_pad, be, wa, ba, wb)


def _edge_body(g_ref, ea_ref, wc_ref, w1_ref, b1_ref, w2r_ref,
               b2_ref, e_ref):
    c = jnp.dot(ea_ref[...], wc_ref[...], preferred_element_type=_F32)
    y0 = jnp.maximum(g_ref[...] + c, 0.0)
    y1 = jnp.maximum(
        jnp.dot(y0, w1_ref[...], preferred_element_type=_F32) + b1_ref[...], 0.0)
    t = jnp.sum(y1 * w2r_ref[...], axis=1, keepdims=True) + b2_ref[0:1, 0:1]
    e_ref[...] = jax.nn.sigmoid(jnp.maximum(t, 0.0))


def _edge(g, ea_pad, wc_pad, w1, b1, w2r, b2b):
    return pl.pallas_call(
        _edge_body,
        grid=(E // _EB,),
        in_specs=[
            pl.BlockSpec((_EB, H), lambda i: (i, 0)),
            pl.BlockSpec((_EB, 8), lambda i: (i, 0)),
            _full((8, H)), _full((H, H)), _full((1, H)), _full((1, H)),
            _full((1, H)),
        ],
        out_specs=pl.BlockSpec((_EB, 1), lambda i: (i, 0)),
        out_shape=jax.ShapeDtypeStruct((E, 1), _F32),
    )(g, ea_pad, wc_pad, w1, b1, w2r, b2b)


def _node_body(h_ref, p0_ref, p1_ref, d0_ref, d1_ref,
               w0a_ref, w0b_ref, w0c_ref, b0_ref, w1_ref, b1_ref,
               w2_ref, b2_ref, wa_ref, ba_ref, wb_ref,
               hn_ref, a_ref, b_ref):
    add = p0_ref[0] + p1_ref[0]
    deg = d0_ref[0] + d1_ref[0]
    mean = add * (1.0 / deg[:, 0:1])
    h = h_ref[...]
    ni = (jnp.dot(h, w0a_ref[...], preferred_element_type=_F32)
          + jnp.dot(add, w0b_ref[...], preferred_element_type=_F32)
          + jnp.dot(mean, w0c_ref[...], preferred_element_type=_F32)
          + b0_ref[...])
    y = jnp.maximum(ni, 0.0)
    y = jnp.maximum(jnp.dot(y, w1_ref[...], preferred_element_type=_F32)
                    + b1_ref[...], 0.0)
    y = jnp.maximum(jnp.dot(y, w2_ref[...], preferred_element_type=_F32)
                    + b2_ref[...], 0.0)
    hn = y + h
    hn_ref[...] = hn
    a_ref[...] = jnp.dot(hn, wa_ref[...], preferred_element_type=_F32) + ba_ref[...]
    b_ref[...] = jnp.dot(hn, wb_ref[...], preferred_element_type=_F32)


def _node(h, parts, deg2, w0a, w0b, w0c, b0, w1, b1, w2, b2, wa, ba, wb):
    return pl.pallas_call(
        _node_body,
        grid=(N // _NB,),
        in_specs=[
            pl.BlockSpec((_NB, H), lambda i: (i, 0)),
            pl.BlockSpec((1, _NB, H), lambda i: (0, i, 0)),
            pl.BlockSpec((1, _NB, H), lambda i: (1, i, 0)),
            pl.BlockSpec((1, _NB, 16), lambda i: (0, i, 0)),
            pl.BlockSpec((1, _NB, 16), lambda i: (1, i, 0)),
            _full((H, H)), _full((H, H)), _full((H, H)), _full((1, H)),
            _full((H, H)), _full((1, H)), _full((H, H)), _full((1, H)),
            _full((H, H)), _full((1, H)), _full((H, H)),
        ],
        out_specs=[pl.BlockSpec((_NB, H), lambda i: (i, 0))] * 3,
        out_shape=[jax.ShapeDtypeStruct((N, H), _F32)] * 3,
    )(h, parts, parts, deg2, deg2, w0a, w0b, w0c, b0, w1, b1, w2, b2, wa, ba, wb)


def _out_body(h_ref, p0_ref, p1_ref, d0_ref, d1_ref,
              w0a_ref, w0b_ref, w0c_ref, b0_ref, w1_ref, b1_ref,
              w2r_ref, b2_ref, o_ref):
    add = p0_ref[0] + p1_ref[0]
    deg = d0_ref[0] + d1_ref[0]
    mean = add * (1.0 / deg[:, 0:1])
    h = h_ref[...]
    ni = (jnp.dot(h, w0a_ref[...], preferred_element_type=_F32)
          + jnp.dot(add, w0b_ref[...], preferred_element_type=_F32)
          + jnp.dot(mean, w0c_ref[...], preferred_element_type=_F32)
          + b0_ref[...])
    y = jnp.maximum(ni, 0.0)
    y = jnp.maximum(jnp.dot(y, w1_ref[...], preferred_element_type=_F32)
                    + b1_ref[...], 0.0)
    t = jnp.sum(y * w2r_ref[...], axis=1, keepdims=True) + b2_ref[0:1, 0:1]
    o_ref[...] = jax.nn.sigmoid(t)


def _out(h, parts, deg2, w0a, w0b, w0c, b0, w1, b1, w2r, b2b):
    return pl.pallas_call(
        _out_body,
        grid=(N // _NB,),
        in_specs=[
            pl.BlockSpec((_NB, H), lambda i: (i, 0)),
            pl.BlockSpec((1, _NB, H), lambda i: (0, i, 0)),
            pl.BlockSpec((1, _NB, H), lambda i: (1, i, 0)),
            pl.BlockSpec((1, _NB, 16), lambda i: (0, i, 0)),
            pl.BlockSpec((1, _NB, 16), lambda i: (1, i, 0)),
            _full((H, H)), _full((H, H)), _full((H, H)), _full((1, H)),
            _full((H, H)), _full((1, H)), _full((1, H)), _full((1, H)),
        ],
        out_specs=pl.BlockSpec((_NB, 1), lambda i: (i, 0)),
        out_shape=jax.ShapeDtypeStruct((N, 1), _F32),
    )(h, parts, parts, deg2, deg2, w0a, w0b, w0c, b0, w1, b1, w2r, b2b)


def kernel(x, edge_index, edge_attr, W_enc, b_enc,
           eW00, eb00, eW01, eb01, eW02, eb02,
           eW10, eb10, eW11, eb11, eW12, eb12,
           eW20, eb20, eW21, eb21, eW22, eb22,
           nW00, nb00, nW01, nb01, nW02, nb02,
           nW10, nb10, nW11, nb11, nW12, nb12,
           oW0, ob0, oW1, ob1, oW2, ob2):
    start = edge_index[0]
    end = edge_index[1]
    x_pad = jnp.pad(x, ((0, 0), (0, 5)))
    we_pad = jnp.pad(W_enc, ((0, 5), (0, 0)))
    ea_pad = jnp.pad(edge_attr, ((0, 0), (0, 6)))
    zeros_nh = jnp.zeros((N, H), _F32)
    zeros_n16 = jnp.zeros((N, 16), _F32)

    eW0 = [eW00, eW10, eW20]
    eb0 = [eb00, eb10, eb20]
    eW1 = [eW01, eW11, eW21]
    eb1 = [eb01, eb11, eb21]
    eW2 = [eW02, eW12, eW22]
    eb2 = [eb02, eb12, eb22]
    nW = [[nW00, nW01, nW02], [nW10, nW11, nW12], [oW0, oW1, oW2]]
    nb = [[nb00, nb01, nb02], [nb10, nb11, nb12], [ob0, ob1, ob2]]

    def w0_split(i):
        w0 = eW0[i]
        return (w0[:H], w0[H:2 * H],
                jnp.pad(w0[2 * H:], ((0, 6), (0, 0))),
                eb0[i].reshape(1, H))

    deg2 = _degree(start, zeros_n16)

    wa0, wb0, _, ba0 = w0_split(0)
    h, a, b = _prep(x_pad, we_pad, b_enc.reshape(1, H), wa0, ba0, wb0)

    out = None
    for i in range(3):
        _, _, wc_pad, _ = w0_split(i)
        w2r = eW2[i].reshape(1, H)
        b2b = jnp.broadcast_to(eb2[i].reshape(1, 1), (1, H))
        g = _gather_sum(a, b, start, end)
        ev = _edge(g, ea_pad, wc_pad, eW1[i], eb1[i].reshape(1, H), w2r, b2b)
        parts = _scatter(h, start, end, ev.reshape(E), zeros_nh)
        w0, w1, w2 = nW[i]
        c0, c1, c2 = nb[i]
        n0a, n0b, n0c = w0[:H], w0[H:2 * H], w0[2 * H:]
        if i < 2:
            wan, wbn, _, ban = w0_split(i + 1)
            h, a, b = _node(h, parts, deg2, n0a, n0b, n0c,
                            c0.reshape(1, H), w1, c1.reshape(1, H),
                            w2, c2.reshape(1, H), wan, ban, wbn)
        else:
            w2r_o = w2.reshape(1, H)
            b2b_o = jnp.broadcast_to(c2.reshape(1, 1), (1, H))
            out = _out(h, parts, deg2, n0a, n0b, n0c,
                       c0.reshape(1, H), w1, c1.reshape(1, H), w2r_o, b2b_o)
    return out
